# Initial kernel scaffold; baseline (speedup 1.0000x reference)
import jax
import jax.numpy as jnp
from jax.experimental import pallas as pl


def kernel(x, P_hat, proj_a_w, proj_b_w, dw_b_w, fi_align_w, router_w, router_b, expert_w1, expert_w2):
    raise NotImplementedError("write your pallas kernel here")


# speedup vs baseline: 5.0791x; 5.0791x over previous
import jax
import jax.numpy as jnp
from jax.experimental import pallas as pl
from jax.experimental.pallas import tpu as pltpu

C = 96
H = W = 224
Hp = Wp = 226
L = Hp * Wp
CH = 6400
NCJ = 9
LTOT = CH * NCJ
NCO = 8
E = 8
TOPK = 2
HALO = Wp + 1
SHIFTS = tuple(dy * Wp + dx for dy in range(3) for dx in range(3))


def _interior_mask(start, length):
    g = start + jax.lax.broadcasted_iota(jnp.int32, (1, length), 1)
    g = jnp.maximum(g, 0)
    row = g // Wp
    col = g - row * Wp
    return ((row >= 1) & (row <= H) & (col >= 1) & (col <= W)).astype(jnp.float32)


def _pre_kernel(ph_ref, xm_ref, xc_ref, xp_ref, aw_ref, bw_ref, dww_ref, al_ref,
                rw_ref, rb_ref, fx_ref, sc_ref, gap_ref):
    jj = pl.program_id(1)
    ext = jnp.concatenate(
        [xm_ref[0, :, CH - HALO:], xc_ref[0], xp_ref[0, :, :HALO]], axis=1)
    xb_ext = jnp.dot(bw_ref[...], ext, preferred_element_type=jnp.float32)
    y = None
    for t, s in enumerate(SHIFTS):
        term = dww_ref[:, t:t + 1] * xb_ext[:, s:s + CH]
        y = term if y is None else y + term
    p_avg = jnp.mean(ph_ref[0], axis=0, keepdims=True)
    q = jnp.dot(p_avg, aw_ref[...], preferred_element_type=jnp.float32)
    fi_s = jnp.dot(q, xc_ref[0], preferred_element_type=jnp.float32)
    fx = (y + al_ref[...] * fi_s) * _interior_mask(jj * CH, CH)
    fx_ref[0] = fx
    part = jnp.sum(fx, axis=1, keepdims=True)

    @pl.when(jj == 0)
    def _():
        gap_ref[...] = part

    @pl.when(jj > 0)
    def _():
        gap_ref[...] = gap_ref[...] + part

    @pl.when(jj == NCJ - 1)
    def _():
        gap = gap_ref[...] * (1.0 / (H * W))
        sc_ref[0] = (jnp.dot(rw_ref[...], gap, preferred_element_type=jnp.float32)
                     + rb_ref[...])


def _route_kernel(sc_ref, idx_ref, gate_ref):
    iota = jax.lax.broadcasted_iota(jnp.int32, (E, 1), 0)
    idxs, gates = [], []
    for b in range(2):
        s = sc_ref[b]
        m1 = jnp.max(s, axis=0, keepdims=True)
        i1 = jnp.min(jnp.where(s == m1, iota, E), axis=0, keepdims=True)
        s2 = jnp.where(iota == i1, jnp.full_like(s, -jnp.inf), s)
        m2 = jnp.max(s2, axis=0, keepdims=True)
        i2 = jnp.min(jnp.where(s2 == m2, iota, E), axis=0, keepdims=True)
        g1 = 1.0 / (1.0 + jnp.exp(m2 - m1))
        idxs += [i1, i2]
        gates += [g1, 1.0 - g1]
    pad_i = jnp.zeros((4, 1), jnp.int32)
    pad_f = jnp.zeros((4, 1), jnp.float32)
    idx_ref[...] = jnp.concatenate(idxs + [pad_i], axis=0)
    gate_ref[...] = jnp.concatenate(gates + [pad_f], axis=0)


def _moe_kernel(idx_ref, gate_ref, fxm_ref, fxc_ref, fxp_ref, w1_ref, w2_ref,
                out_ref):
    b = pl.program_id(0)
    jj = pl.program_id(1)
    k = pl.program_id(2)
    HL = CH + 2 * HALO
    ext = jnp.concatenate(
        [fxm_ref[0, :, CH - 2 * HALO:], fxc_ref[0], fxp_ref[0, :, :2 * HALO]],
        axis=1)
    acc = None
    for t, s in enumerate(SHIFTS):
        term = jnp.dot(w1_ref[0, t], ext[:, s:s + HL],
                       preferred_element_type=jnp.float32)
        acc = term if acc is None else acc + term
    gelu = acc * 0.5 * (1.0 + jax.lax.erf(acc * (2.0 ** -0.5)))
    hmid = gelu * _interior_mask(jj * CH - HALO, HL)
    acc2 = None
    for t, s in enumerate(SHIFTS):
        term = jnp.dot(w2_ref[0, t], hmid[:, s:s + CH],
                       preferred_element_type=jnp.float32)
        acc2 = term if acc2 is None else acc2 + term
    gval = gate_ref[2 * b + k]

    @pl.when(k == 0)
    def _():
        out_ref[0] = gval * acc2

    @pl.when(k == 1)
    def _():
        out_ref[0] = out_ref[0] + gval * acc2


def kernel(x, P_hat, proj_a_w, proj_b_w, dw_b_w, fi_align_w, router_w, router_b,
           expert_w1, expert_w2):
    B = x.shape[0]
    aw = proj_a_w.reshape(C, C)
    bw = proj_b_w.reshape(C, C)
    dww = dw_b_w.reshape(C, 9)
    al = fi_align_w.reshape(C, 1)
    rb = router_b.reshape(E, 1)
    xpad = jnp.pad(x, ((0, 0), (0, 0), (1, 1), (1, 1))).reshape(B, C, L)
    x_flat = jnp.pad(xpad, ((0, 0), (0, 0), (0, LTOT - L)))

    blk = lambda f: pl.BlockSpec((1, C, CH), f)
    fx, scores = pl.pallas_call(
        _pre_kernel,
        grid=(B, NCJ),
        in_specs=[
            pl.BlockSpec((1, P_hat.shape[1], C), lambda b, j: (b, 0, 0)),
            blk(lambda b, j: (b, 0, jnp.maximum(j - 1, 0))),
            blk(lambda b, j: (b, 0, j)),
            blk(lambda b, j: (b, 0, jnp.minimum(j + 1, NCJ - 1))),
            pl.BlockSpec((C, C), lambda b, j: (0, 0)),
            pl.BlockSpec((C, C), lambda b, j: (0, 0)),
            pl.BlockSpec((C, 9), lambda b, j: (0, 0)),
            pl.BlockSpec((C, 1), lambda b, j: (0, 0)),
            pl.BlockSpec((E, C), lambda b, j: (0, 0)),
            pl.BlockSpec((E, 1), lambda b, j: (0, 0)),
        ],
        out_specs=[
            blk(lambda b, j: (b, 0, j)),
            pl.BlockSpec((1, E, 1), lambda b, j: (b, 0, 0)),
        ],
        out_shape=[
            jax.ShapeDtypeStruct((B, C, LTOT), jnp.float32),
            jax.ShapeDtypeStruct((B, E, 1), jnp.float32),
        ],
        scratch_shapes=[pltpu.VMEM((C, 1), jnp.float32)],
        compiler_params=pltpu.CompilerParams(
            dimension_semantics=("arbitrary", "arbitrary")),
    )(P_hat, x_flat, x_flat, x_flat, aw, bw, dww, al, router_w, rb)

    idx8, gate8 = pl.pallas_call(
        _route_kernel,
        out_shape=[
            jax.ShapeDtypeStruct((8, 1), jnp.int32),
            jax.ShapeDtypeStruct((8, 1), jnp.float32),
        ],
    )(scores)
    idx = idx8[:2 * TOPK, 0]
    gate = gate8[:2 * TOPK, 0]

    w1r = expert_w1.transpose(0, 3, 4, 1, 2).reshape(E, 9, C, C)
    w2r = expert_w2.transpose(0, 3, 4, 1, 2).reshape(E, 9, C, C)

    wblk = lambda f: pl.BlockSpec((1, 9, C, C), f)
    grid_spec = pltpu.PrefetchScalarGridSpec(
        num_scalar_prefetch=2,
        grid=(B, NCO, TOPK),
        in_specs=[
            blk(lambda b, j, k, idx, gate: (b, 0, jnp.maximum(j - 1, 0))),
            blk(lambda b, j, k, idx, gate: (b, 0, j)),
            blk(lambda b, j, k, idx, gate: (b, 0, jnp.minimum(j + 1, NCJ - 1))),
            wblk(lambda b, j, k, idx, gate: (idx[2 * b + k], 0, 0, 0)),
            wblk(lambda b, j, k, idx, gate: (idx[2 * b + k], 0, 0, 0)),
        ],
        out_specs=pl.BlockSpec((1, C, CH), lambda b, j, k, idx, gate: (b, 0, j)),
    )
    out_flat = pl.pallas_call(
        _moe_kernel,
        grid_spec=grid_spec,
        out_shape=jax.ShapeDtypeStruct((B, C, NCO * CH), jnp.float32),
        compiler_params=pltpu.CompilerParams(
            dimension_semantics=("arbitrary", "arbitrary", "arbitrary")),
    )(idx, gate, fx, fx, fx, w1r, w2r)

    out = out_flat[:, :, :L].reshape(B, C, Hp, Wp)[:, :, 1:1 + H, 1:1 + W]
    return out + x


# speedup vs baseline: 5.3312x; 1.0496x over previous
import jax
import jax.numpy as jnp
from jax.experimental import pallas as pl
from jax.experimental.pallas import tpu as pltpu

C = 96
H = W = 224
Hp = 226
WL = 256
L = Hp * WL
CH = 7680
NCJ = 8
LTOT = CH * NCJ
E = 8
TOPK = 2
HALO = WL + 1
SHIFTS = tuple(dy * WL + dx for dy in range(3) for dx in range(3))


def _interior_mask(start, length, dtype):
    g = start + jax.lax.broadcasted_iota(jnp.int32, (1, length), 1)
    g = jnp.maximum(g, 0)
    row = g // WL
    col = g - row * WL
    return ((row >= 1) & (row <= H) & (col >= 1) & (col <= W)).astype(dtype)


def _pre_kernel(ph_ref, xm_ref, xc_ref, xp_ref, aw_ref, bw_ref, dww_ref, al_ref,
                rw_ref, rb_ref, fx_ref, sc_ref, gap_ref):
    jj = pl.program_id(1)
    ext = jnp.concatenate(
        [xm_ref[0, :, CH - HALO:], xc_ref[0], xp_ref[0, :, :HALO]], axis=1)
    xb_ext = jnp.dot(bw_ref[...], ext, preferred_element_type=jnp.float32)
    y = None
    for t, s in enumerate(SHIFTS):
        term = dww_ref[:, t:t + 1] * xb_ext[:, s:s + CH]
        y = term if y is None else y + term
    p_avg = jnp.mean(ph_ref[0], axis=0, keepdims=True)
    q = jnp.dot(p_avg, aw_ref[...], preferred_element_type=jnp.float32)
    fi_s = jnp.dot(q, xc_ref[0], preferred_element_type=jnp.float32)
    fx = (y + al_ref[...] * fi_s) * _interior_mask(jj * CH, CH, jnp.float32)
    fx_ref[0] = fx.astype(jnp.bfloat16)
    part = jnp.sum(fx, axis=1, keepdims=True)

    @pl.when(jj == 0)
    def _():
        gap_ref[...] = part

    @pl.when(jj > 0)
    def _():
        gap_ref[...] = gap_ref[...] + part

    @pl.when(jj == NCJ - 1)
    def _():
        gap = gap_ref[...] * (1.0 / (H * W))
        sc_ref[0] = (jnp.dot(rw_ref[...], gap, preferred_element_type=jnp.float32)
                     + rb_ref[...])


def _route_kernel(sc_ref, idx_ref, gate_ref):
    iota = jax.lax.broadcasted_iota(jnp.int32, (E, 1), 0)
    idxs, gates = [], []
    for b in range(2):
        s = sc_ref[b]
        m1 = jnp.max(s, axis=0, keepdims=True)
        i1 = jnp.min(jnp.where(s == m1, iota, E), axis=0, keepdims=True)
        s2 = jnp.where(iota == i1, jnp.full_like(s, -jnp.inf), s)
        m2 = jnp.max(s2, axis=0, keepdims=True)
        i2 = jnp.min(jnp.where(s2 == m2, iota, E), axis=0, keepdims=True)
        g1 = 1.0 / (1.0 + jnp.exp(m2 - m1))
        idxs += [i1, i2]
        gates += [g1, 1.0 - g1]
    pad_i = jnp.zeros((4, 1), jnp.int32)
    pad_f = jnp.zeros((4, 1), jnp.float32)
    idx_ref[...] = jnp.concatenate(idxs + [pad_i], axis=0)
    gate_ref[...] = jnp.concatenate(gates + [pad_f], axis=0)


def _moe_kernel(idx_ref, gate_ref, fxm_ref, fxc_ref, fxp_ref, w1_ref, w2_ref,
                out_ref):
    b = pl.program_id(0)
    jj = pl.program_id(1)
    k = pl.program_id(2)
    HL = CH + 2 * HALO
    ext = jnp.concatenate(
        [fxm_ref[0, :, CH - 2 * HALO:], fxc_ref[0], fxp_ref[0, :, :2 * HALO]],
        axis=1)
    acc = None
    for t, s in enumerate(SHIFTS):
        term = jnp.dot(w1_ref[0, t], ext[:, s:s + HL],
                       preferred_element_type=jnp.float32)
        acc = term if acc is None else acc + term
    gelu = acc * 0.5 * (1.0 + jax.lax.erf(acc * (2.0 ** -0.5)))
    hmid = (gelu * _interior_mask(jj * CH - HALO, HL, jnp.float32)
            ).astype(jnp.bfloat16)
    acc2 = None
    for t, s in enumerate(SHIFTS):
        term = jnp.dot(w2_ref[0, t], hmid[:, s:s + CH],
                       preferred_element_type=jnp.float32)
        acc2 = term if acc2 is None else acc2 + term
    gval = gate_ref[2 * b + k]

    @pl.when(k == 0)
    def _():
        out_ref[0] = gval * acc2

    @pl.when(k == 1)
    def _():
        out_ref[0] = out_ref[0] + gval * acc2


def kernel(x, P_hat, proj_a_w, proj_b_w, dw_b_w, fi_align_w, router_w, router_b,
           expert_w1, expert_w2):
    B = x.shape[0]
    aw = proj_a_w.reshape(C, C)
    bw = proj_b_w.reshape(C, C)
    dww = dw_b_w.reshape(C, 9)
    al = fi_align_w.reshape(C, 1)
    rb = router_b.reshape(E, 1)
    xpad = jnp.pad(x, ((0, 0), (0, 0), (1, 1), (1, WL - 1 - W))).reshape(B, C, L)
    x_flat = jnp.pad(xpad, ((0, 0), (0, 0), (0, LTOT - L)))

    blk = lambda f: pl.BlockSpec((1, C, CH), f)
    fx, scores = pl.pallas_call(
        _pre_kernel,
        grid=(B, NCJ),
        in_specs=[
            pl.BlockSpec((1, P_hat.shape[1], C), lambda b, j: (b, 0, 0)),
            blk(lambda b, j: (b, 0, jnp.maximum(j - 1, 0))),
            blk(lambda b, j: (b, 0, j)),
            blk(lambda b, j: (b, 0, jnp.minimum(j + 1, NCJ - 1))),
            pl.BlockSpec((C, C), lambda b, j: (0, 0)),
            pl.BlockSpec((C, C), lambda b, j: (0, 0)),
            pl.BlockSpec((C, 9), lambda b, j: (0, 0)),
            pl.BlockSpec((C, 1), lambda b, j: (0, 0)),
            pl.BlockSpec((E, C), lambda b, j: (0, 0)),
            pl.BlockSpec((E, 1), lambda b, j: (0, 0)),
        ],
        out_specs=[
            blk(lambda b, j: (b, 0, j)),
            pl.BlockSpec((1, E, 1), lambda b, j: (b, 0, 0)),
        ],
        out_shape=[
            jax.ShapeDtypeStruct((B, C, LTOT), jnp.bfloat16),
            jax.ShapeDtypeStruct((B, E, 1), jnp.float32),
        ],
        scratch_shapes=[pltpu.VMEM((C, 1), jnp.float32)],
        compiler_params=pltpu.CompilerParams(
            dimension_semantics=("arbitrary", "arbitrary")),
    )(P_hat, x_flat, x_flat, x_flat, aw, bw, dww, al, router_w, rb)

    idx8, gate8 = pl.pallas_call(
        _route_kernel,
        out_shape=[
            jax.ShapeDtypeStruct((8, 1), jnp.int32),
            jax.ShapeDtypeStruct((8, 1), jnp.float32),
        ],
    )(scores)
    idx = idx8[:2 * TOPK, 0]
    gate = gate8[:2 * TOPK, 0]

    w1r = expert_w1.transpose(0, 3, 4, 1, 2).reshape(E, 9, C, C).astype(jnp.bfloat16)
    w2r = expert_w2.transpose(0, 3, 4, 1, 2).reshape(E, 9, C, C).astype(jnp.bfloat16)

    wblk = lambda f: pl.BlockSpec((1, 9, C, C), f)
    grid_spec = pltpu.PrefetchScalarGridSpec(
        num_scalar_prefetch=2,
        grid=(B, NCJ, TOPK),
        in_specs=[
            blk(lambda b, j, k, idx, gate: (b, 0, jnp.maximum(j - 1, 0))),
            blk(lambda b, j, k, idx, gate: (b, 0, j)),
            blk(lambda b, j, k, idx, gate: (b, 0, jnp.minimum(j + 1, NCJ - 1))),
            wblk(lambda b, j, k, idx, gate: (idx[2 * b + k], 0, 0, 0)),
            wblk(lambda b, j, k, idx, gate: (idx[2 * b + k], 0, 0, 0)),
        ],
        out_specs=pl.BlockSpec((1, C, CH), lambda b, j, k, idx, gate: (b, 0, j)),
    )
    out_flat = pl.pallas_call(
        _moe_kernel,
        grid_spec=grid_spec,
        out_shape=jax.ShapeDtypeStruct((B, C, LTOT), jnp.float32),
        compiler_params=pltpu.CompilerParams(
            dimension_semantics=("arbitrary", "arbitrary", "arbitrary")),
    )(idx, gate, fx, fx, fx, w1r, w2r)

    out = out_flat[:, :, :L].reshape(B, C, Hp, WL)[:, :, 1:1 + H, 1:1 + W]
    return out + x


# speedup vs baseline: 6.9324x; 1.3003x over previous
import jax
import jax.numpy as jnp
from jax.experimental import pallas as pl
from jax.experimental.pallas import tpu as pltpu

C = 96
H = W = 224
WL = 256
CH = 8192
NCJ = 8
LTOT = CH * NCJ
RB = 32
NCO = 7
E = 8
TOPK = 2
HALO = WL + 1
SHIFTS = tuple(dy * WL + dx for dy in range(3) for dx in range(3))


def _interior_mask(start, length, dtype):
    g = start + jax.lax.broadcasted_iota(jnp.int32, (1, length), 1)
    g = jnp.maximum(g, 0)
    row = g // WL
    col = g - row * WL
    return ((row >= 1) & (row <= H) & (col < W)).astype(dtype)


def _pre_kernel(ph_ref, xm_ref, xc_ref, xp_ref, aw_ref, bw_ref, dww_ref, al_ref,
                rw_ref, rb_ref, fx_ref, idx_ref, gate_ref,
                gap_ref, sc_ref, r1_ref, r2_ref):
    b = pl.program_id(0)
    jj = pl.program_id(1)
    ext = jnp.concatenate(
        [xm_ref[0, :, CH - HALO:], xc_ref[0], xp_ref[0, :, :HALO]], axis=1)
    xb_ext = jnp.dot(bw_ref[...], ext, preferred_element_type=jnp.float32)
    lane = jax.lax.broadcasted_iota(jnp.int32, (1, CH + 2 * HALO), 1)
    xb_ext = jnp.where((jj == 0) & (lane == HALO - 1), 0.0, xb_ext)
    r1_ref[...] = xb_ext[:, 1:1 + CH + 2 * WL]
    r2_ref[...] = xb_ext[:, 2:2 + CH + 2 * WL]
    y = None
    for dy in range(3):
        for dx, src in ((0, None), (1, r1_ref), (2, r2_ref)):
            sl = (xb_ext[:, dy * WL:dy * WL + CH] if src is None
                  else src[:, dy * WL:dy * WL + CH])
            term = dww_ref[:, 3 * dy + dx:3 * dy + dx + 1] * sl
            y = term if y is None else y + term
    p_avg = jnp.mean(ph_ref[0], axis=0, keepdims=True)
    q = jnp.dot(p_avg, aw_ref[...], preferred_element_type=jnp.float32)
    fi_s = jnp.dot(q, xc_ref[0], preferred_element_type=jnp.float32)
    fx = (y + al_ref[...] * fi_s) * _interior_mask(jj * CH, CH, jnp.float32)
    fx_ref[0] = fx.astype(jnp.bfloat16)
    part = jnp.sum(fx, axis=1, keepdims=True)

    @pl.when(jj == 0)
    def _():
        gap_ref[...] = part

    @pl.when(jj > 0)
    def _():
        gap_ref[...] = gap_ref[...] + part

    @pl.when(jj == NCJ - 1)
    def _():
        gap = gap_ref[...] * (1.0 / (H * W))
        scores = (jnp.dot(rw_ref[...], gap, preferred_element_type=jnp.float32)
                  + rb_ref[...])
        sc_ref[pl.ds(E * b, E), :] = scores

    @pl.when((jj == NCJ - 1) & (b == pl.num_programs(0) - 1))
    def _():
        iota = jax.lax.broadcasted_iota(jnp.int32, (E, 1), 0)
        idxs, gates = [], []
        for bb in range(2):
            s = sc_ref[E * bb:E * bb + E, :]
            m1 = jnp.max(s, axis=0, keepdims=True)
            i1 = jnp.min(jnp.where(s == m1, iota, E), axis=0, keepdims=True)
            s2 = jnp.where(iota == i1, jnp.full_like(s, -jnp.inf), s)
            m2 = jnp.max(s2, axis=0, keepdims=True)
            i2 = jnp.min(jnp.where(s2 == m2, iota, E), axis=0, keepdims=True)
            g1 = 1.0 / (1.0 + jnp.exp(m2 - m1))
            idxs += [i1, i2]
            gates += [g1, 1.0 - g1]
        pad_i = jnp.zeros((4, 1), jnp.int32)
        pad_f = jnp.zeros((4, 1), jnp.float32)
        idx_ref[...] = jnp.concatenate(idxs + [pad_i], axis=0)
        gate_ref[...] = jnp.concatenate(gates + [pad_f], axis=0)


def _moe_kernel(idx_ref, gate_ref, fxm_ref, fxc_ref, fxp_ref, xres_ref,
                w1_ref, w2_ref, out_ref):
    b = pl.program_id(0)
    jj = pl.program_id(1)
    k = pl.program_id(2)
    HL = CH + 2 * HALO
    ext = jnp.concatenate(
        [fxm_ref[0, :, CH - (2 * HALO - WL):], fxc_ref[0],
         fxp_ref[0, :, :2 * HALO + WL]], axis=1)
    lane = jax.lax.broadcasted_iota(jnp.int32, (1, CH + 4 * HALO), 1)
    ext = jnp.where((jj == 0) & (lane == HALO), jnp.zeros((), jnp.bfloat16), ext)
    acc = None
    for t, s in enumerate(SHIFTS):
        term = jnp.dot(w1_ref[0, t], ext[:, s:s + HL],
                       preferred_element_type=jnp.float32)
        acc = term if acc is None else acc + term
    gelu = acc * 0.5 * (1.0 + jax.lax.erf(acc * (2.0 ** -0.5)))
    hmid = (gelu * _interior_mask(jj * CH - 1, HL, jnp.float32)
            ).astype(jnp.bfloat16)
    acc2 = None
    for t, s in enumerate(SHIFTS):
        term = jnp.dot(w2_ref[0, t], hmid[:, s:s + CH],
                       preferred_element_type=jnp.float32)
        acc2 = term if acc2 is None else acc2 + term
    gval = gate_ref[2 * b + k]
    resh = (gval * acc2).reshape(C, RB, WL)[:, :, :W]

    @pl.when(k == 0)
    def _():
        out_ref[0] = resh

    @pl.when(k == 1)
    def _():
        out_ref[0] = out_ref[0] + resh + xres_ref[0]


def kernel(x, P_hat, proj_a_w, proj_b_w, dw_b_w, fi_align_w, router_w, router_b,
           expert_w1, expert_w2):
    B = x.shape[0]
    aw = proj_a_w.reshape(C, C)
    bw = proj_b_w.reshape(C, C)
    dww = dw_b_w.reshape(C, 9)
    al = fi_align_w.reshape(C, 1)
    rb = router_b.reshape(E, 1)
    x_flat = jnp.pad(x, ((0, 0), (0, 0), (1, LTOT // WL - 1 - H),
                         (0, WL - W))).reshape(B, C, LTOT)

    blk = lambda f: pl.BlockSpec((1, C, CH), f)
    fx, idx8, gate8 = pl.pallas_call(
        _pre_kernel,
        grid=(B, NCJ),
        in_specs=[
            pl.BlockSpec((1, P_hat.shape[1], C), lambda b, j: (b, 0, 0)),
            blk(lambda b, j: (b, 0, jnp.maximum(j - 1, 0))),
            blk(lambda b, j: (b, 0, j)),
            blk(lambda b, j: (b, 0, jnp.minimum(j + 1, NCJ - 1))),
            pl.BlockSpec((C, C), lambda b, j: (0, 0)),
            pl.BlockSpec((C, C), lambda b, j: (0, 0)),
            pl.BlockSpec((C, 9), lambda b, j: (0, 0)),
            pl.BlockSpec((C, 1), lambda b, j: (0, 0)),
            pl.BlockSpec((E, C), lambda b, j: (0, 0)),
            pl.BlockSpec((E, 1), lambda b, j: (0, 0)),
        ],
        out_specs=[
            blk(lambda b, j: (b, 0, j)),
            pl.BlockSpec((2 * TOPK * 2, 1), lambda b, j: (0, 0)),
            pl.BlockSpec((2 * TOPK * 2, 1), lambda b, j: (0, 0)),
        ],
        out_shape=[
            jax.ShapeDtypeStruct((B, C, LTOT), jnp.bfloat16),
            jax.ShapeDtypeStruct((2 * TOPK * 2, 1), jnp.int32),
            jax.ShapeDtypeStruct((2 * TOPK * 2, 1), jnp.float32),
        ],
        scratch_shapes=[
            pltpu.VMEM((C, 1), jnp.float32),
            pltpu.VMEM((2 * E, 1), jnp.float32),
            pltpu.VMEM((C, CH + 2 * WL), jnp.float32),
            pltpu.VMEM((C, CH + 2 * WL), jnp.float32),
        ],
        compiler_params=pltpu.CompilerParams(
            dimension_semantics=("arbitrary", "arbitrary")),
    )(P_hat, x_flat, x_flat, x_flat, aw, bw, dww, al, router_w, rb)
    idx = idx8.reshape(2 * TOPK * 2)
    gate = gate8.reshape(2 * TOPK * 2)

    w1r = expert_w1.transpose(0, 3, 4, 1, 2).reshape(E, 9, C, C).astype(jnp.bfloat16)
    w2r = expert_w2.transpose(0, 3, 4, 1, 2).reshape(E, 9, C, C).astype(jnp.bfloat16)

    wblk = lambda f: pl.BlockSpec((1, 9, C, C), f)
    grid_spec = pltpu.PrefetchScalarGridSpec(
        num_scalar_prefetch=2,
        grid=(B, NCO, TOPK),
        in_specs=[
            blk(lambda b, j, k, idx, gate: (b, 0, jnp.maximum(j - 1, 0))),
            blk(lambda b, j, k, idx, gate: (b, 0, j)),
            blk(lambda b, j, k, idx, gate: (b, 0, jnp.minimum(j + 1, NCJ - 1))),
            pl.BlockSpec((1, C, RB, W), lambda b, j, k, idx, gate: (b, 0, j, 0)),
            wblk(lambda b, j, k, idx, gate: (idx[2 * b + k], 0, 0, 0)),
            wblk(lambda b, j, k, idx, gate: (idx[2 * b + k], 0, 0, 0)),
        ],
        out_specs=pl.BlockSpec((1, C, RB, W),
                               lambda b, j, k, idx, gate: (b, 0, j, 0)),
    )
    out = pl.pallas_call(
        _moe_kernel,
        grid_spec=grid_spec,
        out_shape=jax.ShapeDtypeStruct((B, C, H, W), jnp.float32),
        compiler_params=pltpu.CompilerParams(
            dimension_semantics=("arbitrary", "arbitrary", "arbitrary")),
    )(idx, gate, fx, fx, fx, x, w1r, w2r)
    return out


# speedup vs baseline: 7.9887x; 1.1524x over previous
import jax
import jax.numpy as jnp
from jax.experimental import pallas as pl
from jax.experimental.pallas import tpu as pltpu

C = 96
H = W = 224
WL = 256
CH = 8192
NCJ = 8
LTOT = CH * NCJ
RB = 32
NCO = 7
E = 8
TOPK = 2
HALO = WL + 1
SHIFTS = tuple(dy * WL + dx for dy in range(3) for dx in range(3))


def _interior_mask(start, length, dtype):
    g = start + jax.lax.broadcasted_iota(jnp.int32, (1, length), 1)
    g = jnp.maximum(g, 0)
    row = g // WL
    col = g - row * WL
    return ((row >= 1) & (row <= H) & (col < W)).astype(dtype)


def _pre_kernel(ph_ref, xm_ref, xc_ref, xp_ref, aw_ref, bw_ref, dww_ref, al_ref,
                rw_ref, rb_ref, fx_ref, idx_ref, gate_ref,
                gap_ref, sc_ref, r1_ref, r2_ref):
    b = pl.program_id(0)
    jj = pl.program_id(1)
    ext = jnp.concatenate(
        [xm_ref[0, :, CH - HALO:], xc_ref[0], xp_ref[0, :, :HALO]], axis=1)
    xb_ext = jnp.dot(bw_ref[...], ext, preferred_element_type=jnp.float32)
    lane = jax.lax.broadcasted_iota(jnp.int32, (1, CH + 2 * HALO), 1)
    xb_ext = jnp.where((jj == 0) & (lane == HALO - 1), 0.0, xb_ext)
    r1_ref[...] = xb_ext[:, 1:1 + CH + 2 * WL]
    r2_ref[...] = xb_ext[:, 2:2 + CH + 2 * WL]
    y = None
    for dy in range(3):
        for dx, src in ((0, None), (1, r1_ref), (2, r2_ref)):
            sl = (xb_ext[:, dy * WL:dy * WL + CH] if src is None
                  else src[:, dy * WL:dy * WL + CH])
            term = dww_ref[:, 3 * dy + dx:3 * dy + dx + 1] * sl
            y = term if y is None else y + term
    p_avg = jnp.mean(ph_ref[0], axis=0, keepdims=True)
    q = jnp.dot(p_avg, aw_ref[...], preferred_element_type=jnp.float32)
    fi_s = jnp.dot(q, xc_ref[0], preferred_element_type=jnp.float32)
    fx = (y + al_ref[...] * fi_s) * _interior_mask(jj * CH, CH, jnp.float32)
    fx_ref[0] = fx.astype(jnp.bfloat16)
    part = jnp.sum(fx, axis=1, keepdims=True)

    @pl.when(jj == 0)
    def _():
        gap_ref[...] = part

    @pl.when(jj > 0)
    def _():
        gap_ref[...] = gap_ref[...] + part

    @pl.when(jj == NCJ - 1)
    def _():
        gap = gap_ref[...] * (1.0 / (H * W))
        scores = (jnp.dot(rw_ref[...], gap, preferred_element_type=jnp.float32)
                  + rb_ref[...])
        sc_ref[pl.ds(E * b, E), :] = scores

    @pl.when((jj == NCJ - 1) & (b == pl.num_programs(0) - 1))
    def _():
        iota = jax.lax.broadcasted_iota(jnp.int32, (E, 1), 0)
        idxs, gates = [], []
        for bb in range(2):
            s = sc_ref[E * bb:E * bb + E, :]
            m1 = jnp.max(s, axis=0, keepdims=True)
            i1 = jnp.min(jnp.where(s == m1, iota, E), axis=0, keepdims=True)
            s2 = jnp.where(iota == i1, jnp.full_like(s, -jnp.inf), s)
            m2 = jnp.max(s2, axis=0, keepdims=True)
            i2 = jnp.min(jnp.where(s2 == m2, iota, E), axis=0, keepdims=True)
            g1 = 1.0 / (1.0 + jnp.exp(m2 - m1))
            idxs += [i1, i2]
            gates += [g1, 1.0 - g1]
        pad_i = jnp.zeros((4, 1), jnp.int32)
        pad_f = jnp.zeros((4, 1), jnp.float32)
        idx_ref[...] = jnp.concatenate(idxs + [pad_i], axis=0)
        gate_ref[...] = jnp.concatenate(gates + [pad_f], axis=0)


def _moe_kernel(idx_ref, gate_ref, fxm_ref, fxc_ref, fxp_ref, xres_ref,
                w1a_ref, w1b_ref, w2a_ref, w2b_ref, out_ref):
    b = pl.program_id(0)
    jj = pl.program_id(1)
    HL = CH + 2 * HALO
    ext = jnp.concatenate(
        [fxm_ref[0, :, CH - (2 * HALO - WL):], fxc_ref[0],
         fxp_ref[0, :, :2 * HALO + WL]], axis=1)
    lane = jax.lax.broadcasted_iota(jnp.int32, (1, CH + 4 * HALO), 1)
    ext = jnp.where((jj == 0) & (lane == HALO), jnp.zeros((), jnp.bfloat16), ext)
    w1cat = jnp.concatenate([w1a_ref[0], w1b_ref[0]], axis=1)
    acc = None
    for t, s in enumerate(SHIFTS):
        term = jnp.dot(w1cat[t], ext[:, s:s + HL],
                       preferred_element_type=jnp.float32)
        acc = term if acc is None else acc + term
    gelu = acc * 0.5 * (1.0 + jax.lax.erf(acc * (2.0 ** -0.5)))
    hmask = _interior_mask(jj * CH - 1, HL, jnp.float32)
    hmid = (gelu * hmask).astype(jnp.bfloat16)
    g0 = gate_ref[2 * b]
    g1 = gate_ref[2 * b + 1]
    w2cat = jnp.concatenate(
        [(w2a_ref[0].astype(jnp.float32) * g0).astype(jnp.bfloat16),
         (w2b_ref[0].astype(jnp.float32) * g1).astype(jnp.bfloat16)],
        axis=2)
    acc2 = None
    for t, s in enumerate(SHIFTS):
        term = jnp.dot(w2cat[t], hmid[:, s:s + CH],
                       preferred_element_type=jnp.float32)
        acc2 = term if acc2 is None else acc2 + term
    out_ref[0] = acc2.reshape(C, RB, WL)[:, :, :W] + xres_ref[0]


def kernel(x, P_hat, proj_a_w, proj_b_w, dw_b_w, fi_align_w, router_w, router_b,
           expert_w1, expert_w2):
    B = x.shape[0]
    aw = proj_a_w.reshape(C, C)
    bw = proj_b_w.reshape(C, C)
    dww = dw_b_w.reshape(C, 9)
    al = fi_align_w.reshape(C, 1)
    rb = router_b.reshape(E, 1)
    x_flat = jnp.pad(x, ((0, 0), (0, 0), (1, LTOT // WL - 1 - H),
                         (0, WL - W))).reshape(B, C, LTOT)

    blk = lambda f: pl.BlockSpec((1, C, CH), f)
    fx, idx8, gate8 = pl.pallas_call(
        _pre_kernel,
        grid=(B, NCJ),
        in_specs=[
            pl.BlockSpec((1, P_hat.shape[1], C), lambda b, j: (b, 0, 0)),
            blk(lambda b, j: (b, 0, jnp.maximum(j - 1, 0))),
            blk(lambda b, j: (b, 0, j)),
            blk(lambda b, j: (b, 0, jnp.minimum(j + 1, NCJ - 1))),
            pl.BlockSpec((C, C), lambda b, j: (0, 0)),
            pl.BlockSpec((C, C), lambda b, j: (0, 0)),
            pl.BlockSpec((C, 9), lambda b, j: (0, 0)),
            pl.BlockSpec((C, 1), lambda b, j: (0, 0)),
            pl.BlockSpec((E, C), lambda b, j: (0, 0)),
            pl.BlockSpec((E, 1), lambda b, j: (0, 0)),
        ],
        out_specs=[
            blk(lambda b, j: (b, 0, j)),
            pl.BlockSpec((2 * TOPK * 2, 1), lambda b, j: (0, 0)),
            pl.BlockSpec((2 * TOPK * 2, 1), lambda b, j: (0, 0)),
        ],
        out_shape=[
            jax.ShapeDtypeStruct((B, C, LTOT), jnp.bfloat16),
            jax.ShapeDtypeStruct((2 * TOPK * 2, 1), jnp.int32),
            jax.ShapeDtypeStruct((2 * TOPK * 2, 1), jnp.float32),
        ],
        scratch_shapes=[
            pltpu.VMEM((C, 1), jnp.float32),
            pltpu.VMEM((2 * E, 1), jnp.float32),
            pltpu.VMEM((C, CH + 2 * WL), jnp.float32),
            pltpu.VMEM((C, CH + 2 * WL), jnp.float32),
        ],
        compiler_params=pltpu.CompilerParams(
            dimension_semantics=("arbitrary", "arbitrary")),
    )(P_hat, x_flat, x_flat, x_flat, aw, bw, dww, al, router_w, rb)
    idx = idx8.reshape(2 * TOPK * 2)
    gate = gate8.reshape(2 * TOPK * 2)

    w1r = expert_w1.transpose(0, 3, 4, 1, 2).reshape(E, 9, C, C).astype(jnp.bfloat16)
    w2r = expert_w2.transpose(0, 3, 4, 1, 2).reshape(E, 9, C, C).astype(jnp.bfloat16)

    wblk = lambda f: pl.BlockSpec((1, 9, C, C), f)
    grid_spec = pltpu.PrefetchScalarGridSpec(
        num_scalar_prefetch=2,
        grid=(B, NCO),
        in_specs=[
            blk(lambda b, j, idx, gate: (b, 0, jnp.maximum(j - 1, 0))),
            blk(lambda b, j, idx, gate: (b, 0, j)),
            blk(lambda b, j, idx, gate: (b, 0, jnp.minimum(j + 1, NCJ - 1))),
            pl.BlockSpec((1, C, RB, W), lambda b, j, idx, gate: (b, 0, j, 0)),
            wblk(lambda b, j, idx, gate: (idx[2 * b], 0, 0, 0)),
            wblk(lambda b, j, idx, gate: (idx[2 * b + 1], 0, 0, 0)),
            wblk(lambda b, j, idx, gate: (idx[2 * b], 0, 0, 0)),
            wblk(lambda b, j, idx, gate: (idx[2 * b + 1], 0, 0, 0)),
        ],
        out_specs=pl.BlockSpec((1, C, RB, W),
                               lambda b, j, idx, gate: (b, 0, j, 0)),
    )
    out = pl.pallas_call(
        _moe_kernel,
        grid_spec=grid_spec,
        out_shape=jax.ShapeDtypeStruct((B, C, H, W), jnp.float32),
        compiler_params=pltpu.CompilerParams(
            dimension_semantics=("arbitrary", "arbitrary")),
    )(idx, gate, fx, fx, fx, x, w1r, w1r, w2r, w2r)
    return out


# speedup vs baseline: 9.1192x; 1.1415x over previous
import jax
import jax.numpy as jnp
from jax.experimental import pallas as pl
from jax.experimental.pallas import tpu as pltpu

C = 96
H = W = 224
WL = 256
CH = 8192
NCJ = 8
LTOT = CH * NCJ
RB = 32
NCO = 7
E = 8
TOPK = 2
HALO = WL + 1
SHIFTS = tuple(dy * WL + dx for dy in range(3) for dx in range(3))


def _interior_mask(start, length, dtype):
    g = start + jax.lax.broadcasted_iota(jnp.int32, (1, length), 1)
    g = jnp.maximum(g, 0)
    row = g // WL
    col = g - row * WL
    return ((row >= 1) & (row <= H) & (col < W)).astype(dtype)


def _pre_kernel(ph_ref, xm_ref, xc_ref, xp_ref, aw_ref, bw_ref, dww_ref, al_ref,
                rw_ref, rb_ref, fx_ref, idx_ref, gate_ref,
                gap_ref, sc_ref, r1_ref, r2_ref):
    b = pl.program_id(0)
    jj = pl.program_id(1)
    ext = jnp.concatenate(
        [xm_ref[0, :, CH - HALO:], xc_ref[0], xp_ref[0, :, :HALO]], axis=1)
    xb_ext = jnp.dot(bw_ref[...], ext, preferred_element_type=jnp.float32)
    lane = jax.lax.broadcasted_iota(jnp.int32, (1, CH + 2 * HALO), 1)
    xb_ext = jnp.where((jj == 0) & (lane == HALO - 1), 0.0, xb_ext)
    r1_ref[...] = xb_ext[:, 1:1 + CH + 2 * WL]
    r2_ref[...] = xb_ext[:, 2:2 + CH + 2 * WL]
    y = None
    for dy in range(3):
        for dx, src in ((0, None), (1, r1_ref), (2, r2_ref)):
            sl = (xb_ext[:, dy * WL:dy * WL + CH] if src is None
                  else src[:, dy * WL:dy * WL + CH])
            term = dww_ref[:, 3 * dy + dx:3 * dy + dx + 1] * sl
            y = term if y is None else y + term
    p_avg = jnp.mean(ph_ref[0], axis=0, keepdims=True)
    q = jnp.dot(p_avg, aw_ref[...], preferred_element_type=jnp.float32)
    fi_s = jnp.dot(q, xc_ref[0], preferred_element_type=jnp.float32)
    fx = (y + al_ref[...] * fi_s) * _interior_mask(jj * CH, CH, jnp.float32)
    fx_ref[0] = fx.astype(jnp.bfloat16)
    part = jnp.sum(fx, axis=1, keepdims=True)

    @pl.when(jj == 0)
    def _():
        gap_ref[...] = part

    @pl.when(jj > 0)
    def _():
        gap_ref[...] = gap_ref[...] + part

    @pl.when(jj == NCJ - 1)
    def _():
        gap = gap_ref[...] * (1.0 / (H * W))
        scores = (jnp.dot(rw_ref[...], gap, preferred_element_type=jnp.float32)
                  + rb_ref[...])
        sc_ref[pl.ds(E * b, E), :] = scores

    @pl.when((jj == NCJ - 1) & (b == pl.num_programs(0) - 1))
    def _():
        iota = jax.lax.broadcasted_iota(jnp.int32, (E, 1), 0)
        idxs, gates = [], []
        for bb in range(2):
            s = sc_ref[E * bb:E * bb + E, :]
            m1 = jnp.max(s, axis=0, keepdims=True)
            i1 = jnp.min(jnp.where(s == m1, iota, E), axis=0, keepdims=True)
            s2 = jnp.where(iota == i1, jnp.full_like(s, -jnp.inf), s)
            m2 = jnp.max(s2, axis=0, keepdims=True)
            i2 = jnp.min(jnp.where(s2 == m2, iota, E), axis=0, keepdims=True)
            g1 = 1.0 / (1.0 + jnp.exp(m2 - m1))
            idxs += [i1, i2]
            gates += [g1, 1.0 - g1]
        pad_i = jnp.zeros((4, 1), jnp.int32)
        pad_f = jnp.zeros((4, 1), jnp.float32)
        idx_ref[...] = jnp.concatenate(idxs + [pad_i], axis=0)
        gate_ref[...] = jnp.concatenate(gates + [pad_f], axis=0)


def _moe_kernel(idx_ref, gate_ref, fxm_ref, fxc_ref, fxp_ref, xres_ref,
                w1a_ref, w1b_ref, w2a_ref, w2b_ref, out_ref):
    b = pl.program_id(0)
    jj = pl.program_id(1)
    HL = CH + 2 * HALO
    ext = jnp.concatenate(
        [fxm_ref[0, :, CH - (2 * HALO - WL):], fxc_ref[0],
         fxp_ref[0, :, :2 * HALO + WL]], axis=1)
    lane = jax.lax.broadcasted_iota(jnp.int32, (1, CH + 4 * HALO), 1)
    ext = jnp.where((jj == 0) & (lane == HALO), jnp.zeros((), jnp.bfloat16), ext)
    w1cat = jnp.concatenate([w1a_ref[0], w1b_ref[0]], axis=1)
    SE = CH + 4 * HALO - 2
    stacked = jnp.concatenate(
        [ext[:, 0:SE], ext[:, 1:SE + 1], ext[:, 2:SE + 2]], axis=0)
    acc = None
    for dy in range(3):
        wdy = jnp.concatenate(
            [w1cat[3 * dy], w1cat[3 * dy + 1], w1cat[3 * dy + 2]],
            axis=1)
        term = jnp.dot(wdy, stacked[:, dy * WL:dy * WL + HL],
                       preferred_element_type=jnp.float32)
        acc = term if acc is None else acc + term
    gelu = acc * 0.5 * (1.0 + jax.lax.erf(acc * (2.0 ** -0.5)))
    hmask = _interior_mask(jj * CH - 1, HL, jnp.float32)
    hmid = (gelu * hmask).astype(jnp.bfloat16)
    g0 = gate_ref[2 * b]
    g1 = gate_ref[2 * b + 1]
    w2cat = jnp.concatenate(
        [(w2a_ref[0].astype(jnp.float32) * g0).astype(jnp.bfloat16),
         (w2b_ref[0].astype(jnp.float32) * g1).astype(jnp.bfloat16)],
        axis=2)
    acc2 = None
    for t, s in enumerate(SHIFTS):
        term = jnp.dot(w2cat[t], hmid[:, s:s + CH],
                       preferred_element_type=jnp.float32)
        acc2 = term if acc2 is None else acc2 + term
    out_ref[0] = acc2.reshape(C, RB, WL)[:, :, :W] + xres_ref[0]


def kernel(x, P_hat, proj_a_w, proj_b_w, dw_b_w, fi_align_w, router_w, router_b,
           expert_w1, expert_w2):
    B = x.shape[0]
    aw = proj_a_w.reshape(C, C)
    bw = proj_b_w.reshape(C, C)
    dww = dw_b_w.reshape(C, 9)
    al = fi_align_w.reshape(C, 1)
    rb = router_b.reshape(E, 1)
    x_flat = jnp.pad(x, ((0, 0), (0, 0), (1, LTOT // WL - 1 - H),
                         (0, WL - W))).reshape(B, C, LTOT)

    blk = lambda f: pl.BlockSpec((1, C, CH), f)
    fx, idx8, gate8 = pl.pallas_call(
        _pre_kernel,
        grid=(B, NCJ),
        in_specs=[
            pl.BlockSpec((1, P_hat.shape[1], C), lambda b, j: (b, 0, 0)),
            blk(lambda b, j: (b, 0, jnp.maximum(j - 1, 0))),
            blk(lambda b, j: (b, 0, j)),
            blk(lambda b, j: (b, 0, jnp.minimum(j + 1, NCJ - 1))),
            pl.BlockSpec((C, C), lambda b, j: (0, 0)),
            pl.BlockSpec((C, C), lambda b, j: (0, 0)),
            pl.BlockSpec((C, 9), lambda b, j: (0, 0)),
            pl.BlockSpec((C, 1), lambda b, j: (0, 0)),
            pl.BlockSpec((E, C), lambda b, j: (0, 0)),
            pl.BlockSpec((E, 1), lambda b, j: (0, 0)),
        ],
        out_specs=[
            blk(lambda b, j: (b, 0, j)),
            pl.BlockSpec((2 * TOPK * 2, 1), lambda b, j: (0, 0)),
            pl.BlockSpec((2 * TOPK * 2, 1), lambda b, j: (0, 0)),
        ],
        out_shape=[
            jax.ShapeDtypeStruct((B, C, LTOT), jnp.bfloat16),
            jax.ShapeDtypeStruct((2 * TOPK * 2, 1), jnp.int32),
            jax.ShapeDtypeStruct((2 * TOPK * 2, 1), jnp.float32),
        ],
        scratch_shapes=[
            pltpu.VMEM((C, 1), jnp.float32),
            pltpu.VMEM((2 * E, 1), jnp.float32),
            pltpu.VMEM((C, CH + 2 * WL), jnp.float32),
            pltpu.VMEM((C, CH + 2 * WL), jnp.float32),
        ],
        compiler_params=pltpu.CompilerParams(
            dimension_semantics=("arbitrary", "arbitrary")),
    )(P_hat, x_flat, x_flat, x_flat, aw, bw, dww, al, router_w, rb)
    idx = idx8.reshape(2 * TOPK * 2)
    gate = gate8.reshape(2 * TOPK * 2)

    w1r = expert_w1.transpose(0, 3, 4, 1, 2).reshape(E, 9, C, C).astype(jnp.bfloat16)
    w2r = expert_w2.transpose(0, 3, 4, 1, 2).reshape(E, 9, C, C).astype(jnp.bfloat16)

    wblk = lambda f: pl.BlockSpec((1, 9, C, C), f)
    grid_spec = pltpu.PrefetchScalarGridSpec(
        num_scalar_prefetch=2,
        grid=(B, NCO),
        in_specs=[
            blk(lambda b, j, idx, gate: (b, 0, jnp.maximum(j - 1, 0))),
            blk(lambda b, j, idx, gate: (b, 0, j)),
            blk(lambda b, j, idx, gate: (b, 0, jnp.minimum(j + 1, NCJ - 1))),
            pl.BlockSpec((1, C, RB, W), lambda b, j, idx, gate: (b, 0, j, 0)),
            wblk(lambda b, j, idx, gate: (idx[2 * b], 0, 0, 0)),
            wblk(lambda b, j, idx, gate: (idx[2 * b + 1], 0, 0, 0)),
            wblk(lambda b, j, idx, gate: (idx[2 * b], 0, 0, 0)),
            wblk(lambda b, j, idx, gate: (idx[2 * b + 1], 0, 0, 0)),
        ],
        out_specs=pl.BlockSpec((1, C, RB, W),
                               lambda b, j, idx, gate: (b, 0, j, 0)),
    )
    out = pl.pallas_call(
        _moe_kernel,
        grid_spec=grid_spec,
        out_shape=jax.ShapeDtypeStruct((B, C, H, W), jnp.float32),
        compiler_params=pltpu.CompilerParams(
            dimension_semantics=("parallel", "parallel")),
    )(idx, gate, fx, fx, fx, x, w1r, w1r, w2r, w2r)
    return out
